# baseline (speedup 1.0000x reference)
import jax
import jax.numpy as jnp
from jax.experimental import pallas as pl
from jax.experimental.pallas import tpu as pltpu

_NP = 98
_NPP = 128
_HID = 128
_NL = 4
_INCH = 256
_HW = 32 * 32
_NB = 8

_F = jnp.float32


def _dot(a, b):
    return jnp.dot(a, b, preferred_element_type=_F)


def _fused_kernel(x_ref, cw_ref, cb_ref, win_ref, bin_ref,
                  wq_ref, wk_ref, wg_ref, bg_ref, wout_ref, bout_ref,
                  out_ref):
    scale = 1.0 / jnp.sqrt(_F(_HID))
    col = jax.lax.broadcasted_iota(jnp.int32, (_NPP, _NPP), 1)
    colmask = col < _NP
    hs = [_dot(cw_ref[...], x_ref[s]) for s in range(_NB)]
    hs = [jnp.maximum(h + cb_ref[...], 0.0) for h in hs]
    xs = []
    for s in range(_NB):
        acc = _dot(hs[s][0:_NPP], win_ref[0])
        for j in range(1, 4):
            acc = acc + _dot(hs[s][j * _NPP:(j + 1) * _NPP], win_ref[j])
        xs.append(acc)
    xs = [jnp.maximum(x + bin_ref[...], 0.0) for x in xs]
    for l in range(_NL):
        xcat = jnp.concatenate(xs, axis=0)
        qc = _dot(xcat, wq_ref[l])
        kc = _dot(xcat, wk_ref[l])
        gc = _dot(xcat, wg_ref[l])
        qs = [qc[s * _NPP:(s + 1) * _NPP] for s in range(_NB)]
        ks = [kc[s * _NPP:(s + 1) * _NPP] for s in range(_NB)]
        gs = [gc[s * _NPP:(s + 1) * _NPP] for s in range(_NB)]
        ls_ = [jax.lax.dot_general(q, k, (((1,), (1,)), ((), ())),
                                   preferred_element_type=_F) * scale
               for q, k in zip(qs, ks)]
        ls_ = [jnp.where(colmask, lg, -1e30) for lg in ls_]
        as_ = [jax.nn.softmax(lg, axis=-1) for lg in ls_]
        msgs = [_dot(a, g) + bg_ref[l] for a, g in zip(as_, gs)]
        xs = [jnp.maximum(m + x, 0.0) for m, x in zip(msgs, xs)]
    ocat = _dot(jnp.concatenate(xs, axis=0), wout_ref[...]) + bout_ref[...]
    for s in range(_NB):
        out_ref[s] = ocat[s * _NPP:(s + 1) * _NPP]


def kernel(x_dict, conv_w, conv_b, W_in, b_in, Wq, Wk, Wg, bg, W_out, b_out):
    b = x_dict.shape[0]
    xr = x_dict.reshape(b, _INCH, _HW)
    cw_g = conv_w.reshape(_NP, 4, _INCH).transpose(1, 0, 2)
    cw_p = jnp.zeros((4, _NPP, _INCH), _F).at[:, :_NP].set(cw_g)
    cw_p = cw_p.reshape(4 * _NPP, _INCH)
    cb_g = conv_b.reshape(_NP, 4).T
    cb_p = jnp.zeros((4, _NPP), _F).at[:, :_NP].set(cb_g)
    cb_p = cb_p.reshape(4 * _NPP, 1)
    win_r = W_in.reshape(4, _HW, _HID)
    bin_r = b_in.reshape(1, _HID)
    bg_r = bg.reshape(_NL, 1, _HID)
    wout_p = jnp.zeros((_HID, _HID), _F).at[:, :2].set(W_out)
    bout_p = jnp.zeros((1, _HID), _F).at[0, :2].set(b_out)

    out = pl.pallas_call(
        _fused_kernel,
        grid=(b // _NB,),
        compiler_params=pltpu.CompilerParams(
            dimension_semantics=("parallel",)),
        in_specs=[
            pl.BlockSpec((_NB, _INCH, _HW), lambda i: (i, 0, 0)),
            pl.BlockSpec((4 * _NPP, _INCH), lambda i: (0, 0)),
            pl.BlockSpec((4 * _NPP, 1), lambda i: (0, 0)),
            pl.BlockSpec((4, _HW, _HID), lambda i: (0, 0, 0)),
            pl.BlockSpec((1, _HID), lambda i: (0, 0)),
            pl.BlockSpec((_NL, _HID, _HID), lambda i: (0, 0, 0)),
            pl.BlockSpec((_NL, _HID, _HID), lambda i: (0, 0, 0)),
            pl.BlockSpec((_NL, _HID, _HID), lambda i: (0, 0, 0)),
            pl.BlockSpec((_NL, 1, _HID), lambda i: (0, 0, 0)),
            pl.BlockSpec((_HID, _HID), lambda i: (0, 0)),
            pl.BlockSpec((1, _HID), lambda i: (0, 0)),
        ],
        out_specs=pl.BlockSpec((_NB, _NPP, _HID), lambda i: (i, 0, 0)),
        out_shape=jax.ShapeDtypeStruct((b, _NPP, _HID), jnp.float32),
    )(xr, cw_p, cb_p, win_r, bin_r, Wq, Wk, Wg, bg_r, wout_p, bout_p)
    return out[:, :_NP, :2].reshape(b, -1)
